# baseline (speedup 1.0000x reference)
import jax
import jax.numpy as jnp
from jax.experimental import pallas as pl

_B, _N, _C = 64, 576, 768
_D, _M, _K, _H = 64, 16, 64, 4
_HD = _D // _H
_L = _M + _K


def _adapter_kernel(img_ref, sel_ref, wd_ref, bdown_ref, wu_ref, bup_ref,
                    q_ref, wq_ref, wk_ref, wv_ref, bq_ref, bk_ref, bv_ref,
                    wo_ref, bo_ref, lnw_ref, lnb_ref, gamma_ref, out_ref):
    f32 = jnp.float32
    img = img_ref[0]
    proj = jnp.dot(img, wd_ref[...], preferred_element_type=f32) + bdown_ref[...]
    act = 0.5 * proj * (1.0 + jax.lax.erf(proj * 0.7071067811865476))

    sel = sel_ref[0]
    sel_n = sel / jnp.maximum(jnp.sqrt(jnp.sum(sel * sel)), 1e-12)
    act_norm = jnp.sqrt(jnp.sum(act * act, axis=1, keepdims=True))
    s_col = (jnp.sum(act * sel_n, axis=1, keepdims=True)
             / jnp.maximum(act_norm, 1e-12))

    eye_rows = jax.lax.broadcasted_iota(jnp.int32, (_N, _N), 0)
    eye_cols = jax.lax.broadcasted_iota(jnp.int32, (_N, _N), 1)
    eye = (eye_rows == eye_cols).astype(f32)
    s_row = jax.lax.dot_general(s_col, eye, (((0,), (0,)), ((), ())),
                                preferred_element_type=f32)

    row_i = jax.lax.broadcasted_iota(jnp.int32, (_N, _N), 0)
    col_j = jax.lax.broadcasted_iota(jnp.int32, (_N, _N), 1)
    beats = (s_row > s_col) | ((s_row == s_col) & (col_j < row_i))
    rank = jnp.sum(beats.astype(f32), axis=1, keepdims=True)

    k_iota = jax.lax.broadcasted_iota(jnp.int32, (_N, _K), 1).astype(f32)
    p_t = (rank == k_iota).astype(f32) * (rank < _K).astype(f32)

    sparse = jax.lax.dot_general(p_t, act, (((0,), (0,)), ((), ())),
                                 preferred_element_type=f32)
    comb = jnp.concatenate([q_ref[...], sparse], axis=0)

    mu = jnp.mean(comb, axis=1, keepdims=True)
    var = jnp.mean((comb - mu) ** 2, axis=1, keepdims=True)
    xn = (comb - mu) * jax.lax.rsqrt(var + 1e-5) * lnw_ref[...] + lnb_ref[...]

    q = jnp.dot(xn, wq_ref[...], preferred_element_type=f32) + bq_ref[...]
    k = jnp.dot(xn, wk_ref[...], preferred_element_type=f32) + bk_ref[...]
    v = jnp.dot(xn, wv_ref[...], preferred_element_type=f32) + bv_ref[...]

    lane = jax.lax.broadcasted_iota(jnp.int32, (1, _D), 1)
    att = bo_ref[...]
    o = jnp.zeros((_L, _D), f32)
    for h in range(_H):
        m_h = ((lane // _HD) == h).astype(f32)
        logits = jax.lax.dot_general(q * m_h, k * m_h,
                                     (((1,), (1,)), ((), ())),
                                     preferred_element_type=f32) / 4.0
        logits = logits - jnp.max(logits, axis=1, keepdims=True)
        e = jnp.exp(logits)
        a = e / jnp.sum(e, axis=1, keepdims=True)
        o = o + jnp.dot(a, v * m_h, preferred_element_type=f32)
    att = jnp.dot(o, wo_ref[...], preferred_element_type=f32) + bo_ref[...]

    enh = comb + att
    enh_sparse = enh[_M:, :]

    upd = jnp.dot(enh_sparse, wu_ref[...], preferred_element_type=f32)
    scat = jnp.dot(p_t, upd, preferred_element_type=f32)
    gamma = gamma_ref[0, 0]
    out_ref[0] = img + gamma * (scat + bup_ref[...])


def _run(img, sel, wd_t, bdown, wu_t, bup, queries,
         wq_t, wk_t, wv_t, bq, bk, bv, wo_t, bo, lnw, lnb, gamma):
    grid = (_B,)
    def first(b):
        return (b, 0, 0)
    def whole2(b):
        return (0, 0)
    specs = [
        pl.BlockSpec((1, _N, _C), first),
        pl.BlockSpec((1, 1, _D), first),
        pl.BlockSpec((_C, _D), whole2),
        pl.BlockSpec((1, _D), whole2),
        pl.BlockSpec((_D, _C), whole2),
        pl.BlockSpec((1, _C), whole2),
        pl.BlockSpec((_M, _D), whole2),
        pl.BlockSpec((_D, _D), whole2),
        pl.BlockSpec((_D, _D), whole2),
        pl.BlockSpec((_D, _D), whole2),
        pl.BlockSpec((1, _D), whole2),
        pl.BlockSpec((1, _D), whole2),
        pl.BlockSpec((1, _D), whole2),
        pl.BlockSpec((_D, _D), whole2),
        pl.BlockSpec((1, _D), whole2),
        pl.BlockSpec((1, _D), whole2),
        pl.BlockSpec((1, _D), whole2),
        pl.BlockSpec((1, 1), whole2),
    ]
    return pl.pallas_call(
        _adapter_kernel,
        grid=grid,
        in_specs=specs,
        out_specs=pl.BlockSpec((1, _N, _C), first),
        out_shape=jax.ShapeDtypeStruct((_B, _N, _C), jnp.float32),
    )(img, sel, wd_t, bdown, wu_t, bup, queries,
      wq_t, wk_t, wv_t, bq, bk, bv, wo_t, bo, lnw, lnb, gamma)


def kernel(image_features, text_features, W_down, b_down, W_up, b_up, m_queries,
           in_proj_w, in_proj_b, out_proj_w, out_proj_b, ln_w, ln_b, gamma):
    f32 = jnp.float32
    sel = text_features[:, 0:1, :_D]
    wd_t = W_down.T
    wu_t = W_up.T
    wq_t = in_proj_w[0:_D, :].T
    wk_t = in_proj_w[_D:2 * _D, :].T
    wv_t = in_proj_w[2 * _D:3 * _D, :].T
    bq = in_proj_b[0:_D].reshape(1, _D)
    bk = in_proj_b[_D:2 * _D].reshape(1, _D)
    bv = in_proj_b[2 * _D:3 * _D].reshape(1, _D)
    wo_t = out_proj_w.T
    bo = out_proj_b.reshape(1, _D)
    return _run(image_features, sel, wd_t, b_down.reshape(1, _D), wu_t,
                b_up.reshape(1, _C), m_queries[0], wq_t, wk_t, wv_t,
                bq, bk, bv, wo_t, bo, ln_w.reshape(1, _D),
                ln_b.reshape(1, _D), jnp.asarray(gamma, f32).reshape(1, 1))


# speedup vs baseline: 1.3025x; 1.3025x over previous
import jax
import jax.numpy as jnp
from jax.experimental import pallas as pl

_B, _N, _C = 64, 576, 768
_D, _M, _K, _H = 64, 16, 64, 4
_HD = _D // _H
_L = _M + _K
_BB = 4
_G = _B // _BB


def _adapter_kernel(img_ref, sel_ref, wd_ref, bdown_ref, wu_ref, bup_ref,
                    q_ref, wq_ref, wk_ref, wv_ref, bq_ref, bk_ref, bv_ref,
                    wo_ref, bo_ref, lnw_ref, lnb_ref, gamma_ref, out_ref):
    f32 = jnp.float32
    imgs = img_ref[...].reshape(_BB * _N, _C)
    proj = jnp.dot(imgs, wd_ref[...], preferred_element_type=f32) + bdown_ref[...]
    acts = 0.5 * proj * (1.0 + jax.lax.erf(proj * 0.7071067811865476))
    acts_sq = acts * acts
    gamma = gamma_ref[0, 0]
    ones_d = jnp.ones((1, _D), f32)
    row_i = jax.lax.broadcasted_iota(jnp.int32, (_N, _N), 0)
    col_j = jax.lax.broadcasted_iota(jnp.int32, (_N, _N), 1)
    tie = col_j < row_i
    k_iota = jax.lax.broadcasted_iota(jnp.int32, (_N, _K), 1).astype(f32)
    lane = jax.lax.broadcasted_iota(jnp.int32, (1, _D), 1)

    for bb in range(_BB):
        act = acts[bb * _N:(bb + 1) * _N, :]
        act_sq = acts_sq[bb * _N:(bb + 1) * _N, :]
        sel = sel_ref[bb]
        sel_n = sel / jnp.maximum(jnp.sqrt(jnp.sum(sel * sel)), 1e-12)

        nrm2_c = jnp.sum(act_sq, axis=1, keepdims=True)
        s_col = (jnp.sum(act * sel_n, axis=1, keepdims=True)
                 / jnp.maximum(jnp.sqrt(nrm2_c), 1e-12))
        s_row = jnp.swapaxes(s_col, 0, 1)

        beats = (s_row > s_col) | ((s_row == s_col) & tie)
        rank = jnp.sum(beats.astype(f32), axis=1, keepdims=True)

        p_t = (rank == k_iota).astype(f32) * (rank < _K).astype(f32)

        sparse = jax.lax.dot_general(p_t, act, (((0,), (0,)), ((), ())),
                                     preferred_element_type=f32)
        comb = jnp.concatenate([q_ref[...], sparse], axis=0)

        mu = jnp.mean(comb, axis=1, keepdims=True)
        var = jnp.mean((comb - mu) ** 2, axis=1, keepdims=True)
        xn = ((comb - mu) * jax.lax.rsqrt(var + 1e-5) * lnw_ref[...]
              + lnb_ref[...])

        q = jnp.dot(xn, wq_ref[...], preferred_element_type=f32) + bq_ref[...]
        k = jnp.dot(xn, wk_ref[...], preferred_element_type=f32) + bk_ref[...]
        v = jnp.dot(xn, wv_ref[...], preferred_element_type=f32) + bv_ref[...]

        o = jnp.zeros((_L, _D), f32)
        for h in range(_H):
            m_h = ((lane // _HD) == h).astype(f32)
            logits = jax.lax.dot_general(q * m_h, k * m_h,
                                         (((1,), (1,)), ((), ())),
                                         preferred_element_type=f32) / 4.0
            logits = logits - jnp.max(logits, axis=1, keepdims=True)
            e = jnp.exp(logits)
            a = e / jnp.sum(e, axis=1, keepdims=True)
            o = o + jnp.dot(a, v * m_h, preferred_element_type=f32)
        att = jnp.dot(o, wo_ref[...], preferred_element_type=f32) + bo_ref[...]

        enh = comb + att
        enh_sparse = enh[_M:, :]

        upd = jnp.dot(enh_sparse, wu_ref[...],
                      preferred_element_type=f32)
        scat = jnp.dot(p_t, upd, preferred_element_type=f32)
        out_ref[bb] = img_ref[bb] + gamma * (scat + bup_ref[...])


def _run(img, sel, wd_t, bdown, wu_t, bup, queries,
         wq_t, wk_t, wv_t, bq, bk, bv, wo_t, bo, lnw, lnb, gamma):
    def first(b):
        return (b, 0, 0)
    def whole2(b):
        return (0, 0)
    specs = [
        pl.BlockSpec((_BB, _N, _C), first),
        pl.BlockSpec((_BB, 1, _D), first),
        pl.BlockSpec((_C, _D), whole2),
        pl.BlockSpec((1, _D), whole2),
        pl.BlockSpec((_D, _C), whole2),
        pl.BlockSpec((1, _C), whole2),
        pl.BlockSpec((_M, _D), whole2),
        pl.BlockSpec((_D, _D), whole2),
        pl.BlockSpec((_D, _D), whole2),
        pl.BlockSpec((_D, _D), whole2),
        pl.BlockSpec((1, _D), whole2),
        pl.BlockSpec((1, _D), whole2),
        pl.BlockSpec((1, _D), whole2),
        pl.BlockSpec((_D, _D), whole2),
        pl.BlockSpec((1, _D), whole2),
        pl.BlockSpec((1, _D), whole2),
        pl.BlockSpec((1, _D), whole2),
        pl.BlockSpec((1, 1), whole2),
    ]
    return pl.pallas_call(
        _adapter_kernel,
        grid=(_G,),
        in_specs=specs,
        out_specs=pl.BlockSpec((_BB, _N, _C), first),
        out_shape=jax.ShapeDtypeStruct((_B, _N, _C), jnp.float32),
    )(img, sel, wd_t, bdown, wu_t, bup, queries,
      wq_t, wk_t, wv_t, bq, bk, bv, wo_t, bo, lnw, lnb, gamma)


def kernel(image_features, text_features, W_down, b_down, W_up, b_up, m_queries,
           in_proj_w, in_proj_b, out_proj_w, out_proj_b, ln_w, ln_b, gamma):
    f32 = jnp.float32
    sel = text_features[:, 0:1, :_D]
    wd_t = W_down.T
    wu_t = W_up.T
    wq_t = in_proj_w[0:_D, :].T
    wk_t = in_proj_w[_D:2 * _D, :].T
    wv_t = in_proj_w[2 * _D:3 * _D, :].T
    bq = in_proj_b[0:_D].reshape(1, _D)
    bk = in_proj_b[_D:2 * _D].reshape(1, _D)
    bv = in_proj_b[2 * _D:3 * _D].reshape(1, _D)
    wo_t = out_proj_w.T
    bo = out_proj_b.reshape(1, _D)
    return _run(image_features, sel, wd_t, b_down.reshape(1, _D), wu_t,
                b_up.reshape(1, _C), m_queries[0], wq_t, wk_t, wv_t,
                bq, bk, bv, wo_t, bo, ln_w.reshape(1, _D),
                ln_b.reshape(1, _D), jnp.asarray(gamma, f32).reshape(1, 1))


# speedup vs baseline: 1.3032x; 1.0006x over previous
import jax
import jax.numpy as jnp
from jax.experimental import pallas as pl
from jax.experimental.pallas import tpu as pltpu

_B, _N, _C = 64, 576, 768
_D, _M, _K, _H = 64, 16, 64, 4
_HD = _D // _H
_L = _M + _K
_BB = 4
_G = _B // _BB


def _adapter_kernel(img_ref, sel_ref, wd_ref, bdown_ref, wu_ref, bup_ref,
                    q_ref, wq_ref, wk_ref, wv_ref, bq_ref, bk_ref, bv_ref,
                    wo_ref, bo_ref, lnw_ref, lnb_ref, gamma_ref, out_ref):
    f32 = jnp.float32
    imgs = img_ref[...].reshape(_BB * _N, _C)
    proj = jnp.dot(imgs, wd_ref[...], preferred_element_type=f32) + bdown_ref[...]
    acts = 0.5 * proj * (1.0 + jax.lax.erf(proj * 0.7071067811865476))
    acts_sq = acts * acts
    gamma = gamma_ref[0, 0]
    ones_d = jnp.ones((1, _D), f32)
    row_i = jax.lax.broadcasted_iota(jnp.int32, (_N, _N), 0)
    col_j = jax.lax.broadcasted_iota(jnp.int32, (_N, _N), 1)
    tie = col_j < row_i
    k_iota = jax.lax.broadcasted_iota(jnp.int32, (_N, _K), 1).astype(f32)
    lane = jax.lax.broadcasted_iota(jnp.int32, (1, _D), 1)

    for bb in range(_BB):
        act = acts[bb * _N:(bb + 1) * _N, :]
        act_sq = acts_sq[bb * _N:(bb + 1) * _N, :]
        sel = sel_ref[bb]
        sel_n = sel / jnp.maximum(jnp.sqrt(jnp.sum(sel * sel)), 1e-12)

        nrm2_c = jnp.sum(act_sq, axis=1, keepdims=True)
        s_col = (jnp.sum(act * sel_n, axis=1, keepdims=True)
                 / jnp.maximum(jnp.sqrt(nrm2_c), 1e-12))
        s_row = jnp.swapaxes(s_col, 0, 1)

        beats = (s_row > s_col) | ((s_row == s_col) & tie)
        rank = jnp.sum(beats.astype(f32), axis=1, keepdims=True)

        p_t = (rank == k_iota).astype(f32) * (rank < _K).astype(f32)

        sparse = jax.lax.dot_general(p_t, act, (((0,), (0,)), ((), ())),
                                     preferred_element_type=f32)
        comb = jnp.concatenate([q_ref[...], sparse], axis=0)

        mu = jnp.mean(comb, axis=1, keepdims=True)
        var = jnp.mean((comb - mu) ** 2, axis=1, keepdims=True)
        xn = ((comb - mu) * jax.lax.rsqrt(var + 1e-5) * lnw_ref[...]
              + lnb_ref[...])

        q = jnp.dot(xn, wq_ref[...], preferred_element_type=f32) + bq_ref[...]
        k = jnp.dot(xn, wk_ref[...], preferred_element_type=f32) + bk_ref[...]
        v = jnp.dot(xn, wv_ref[...], preferred_element_type=f32) + bv_ref[...]

        o = jnp.zeros((_L, _D), f32)
        for h in range(_H):
            m_h = ((lane // _HD) == h).astype(f32)
            logits = jax.lax.dot_general(q * m_h, k * m_h,
                                         (((1,), (1,)), ((), ())),
                                         preferred_element_type=f32) / 4.0
            logits = logits - jnp.max(logits, axis=1, keepdims=True)
            e = jnp.exp(logits)
            a = e / jnp.sum(e, axis=1, keepdims=True)
            o = o + jnp.dot(a, v * m_h, preferred_element_type=f32)
        att = jnp.dot(o, wo_ref[...], preferred_element_type=f32) + bo_ref[...]

        enh = comb + att
        enh_sparse = enh[_M:, :]

        upd = jnp.dot(enh_sparse, wu_ref[...],
                      preferred_element_type=f32)
        scat = jnp.dot(p_t, upd, preferred_element_type=f32)
        out_ref[bb] = img_ref[bb] + gamma * (scat + bup_ref[...])


def _run(img, sel, wd_t, bdown, wu_t, bup, queries,
         wq_t, wk_t, wv_t, bq, bk, bv, wo_t, bo, lnw, lnb, gamma):
    def first(b):
        return (b, 0, 0)
    def whole2(b):
        return (0, 0)
    specs = [
        pl.BlockSpec((_BB, _N, _C), first),
        pl.BlockSpec((_BB, 1, _D), first),
        pl.BlockSpec((_C, _D), whole2),
        pl.BlockSpec((1, _D), whole2),
        pl.BlockSpec((_D, _C), whole2),
        pl.BlockSpec((1, _C), whole2),
        pl.BlockSpec((_M, _D), whole2),
        pl.BlockSpec((_D, _D), whole2),
        pl.BlockSpec((_D, _D), whole2),
        pl.BlockSpec((_D, _D), whole2),
        pl.BlockSpec((1, _D), whole2),
        pl.BlockSpec((1, _D), whole2),
        pl.BlockSpec((1, _D), whole2),
        pl.BlockSpec((_D, _D), whole2),
        pl.BlockSpec((1, _D), whole2),
        pl.BlockSpec((1, _D), whole2),
        pl.BlockSpec((1, _D), whole2),
        pl.BlockSpec((1, 1), whole2),
    ]
    return pl.pallas_call(
        _adapter_kernel,
        grid=(_G,),
        in_specs=specs,
        out_specs=pl.BlockSpec((_BB, _N, _C), first),
        out_shape=jax.ShapeDtypeStruct((_B, _N, _C), jnp.float32),
        compiler_params=pltpu.CompilerParams(
            dimension_semantics=("parallel",)),
    )(img, sel, wd_t, bdown, wu_t, bup, queries,
      wq_t, wk_t, wv_t, bq, bk, bv, wo_t, bo, lnw, lnb, gamma)


def kernel(image_features, text_features, W_down, b_down, W_up, b_up, m_queries,
           in_proj_w, in_proj_b, out_proj_w, out_proj_b, ln_w, ln_b, gamma):
    f32 = jnp.float32
    sel = text_features[:, 0:1, :_D]
    wd_t = W_down.T
    wu_t = W_up.T
    wq_t = in_proj_w[0:_D, :].T
    wk_t = in_proj_w[_D:2 * _D, :].T
    wv_t = in_proj_w[2 * _D:3 * _D, :].T
    bq = in_proj_b[0:_D].reshape(1, _D)
    bk = in_proj_b[_D:2 * _D].reshape(1, _D)
    bv = in_proj_b[2 * _D:3 * _D].reshape(1, _D)
    wo_t = out_proj_w.T
    bo = out_proj_b.reshape(1, _D)
    return _run(image_features, sel, wd_t, b_down.reshape(1, _D), wu_t,
                b_up.reshape(1, _C), m_queries[0], wq_t, wk_t, wv_t,
                bq, bk, bv, wo_t, bo, ln_w.reshape(1, _D),
                ln_b.reshape(1, _D), jnp.asarray(gamma, f32).reshape(1, 1))
